# Initial kernel scaffold; baseline (speedup 1.0000x reference)
import jax
import jax.numpy as jnp
from jax.experimental import pallas as pl


def kernel(x, edge_index, edge_attr, batch, W1, We, be, Wn1, bn1, Wn2, bn2, Wfc, bfc):
    raise NotImplementedError("write your pallas kernel here")


# speedup vs baseline: 3.6906x; 3.6906x over previous
import functools

import jax
import jax.numpy as jnp
from jax import lax
from jax.experimental import pallas as pl
from jax.experimental.pallas import tpu as pltpu
from jax.experimental.pallas import tpu_sc as plsc

_N = 10000
_E = 320000
_D = 128
_DEP = 32
_G = 64

_NC = 2
_NS = 16
_NW = _NC * _NS
_EPW = _E // _NW
_CHUNK = 80
_NCHUNK = _EPW // _CHUNK
_NP = 10240
_RPT = _NP // _NS


def _sc_edge_pass(with_ea):
    out_type = [jax.ShapeDtypeStruct((_NC, _NP, _D), jnp.float32)]
    scratch = [
        pltpu.VMEM((_CHUNK,), jnp.int32),
        pltpu.VMEM((_CHUNK,), jnp.int32),
        pltpu.VMEM((_CHUNK, _D), jnp.float32),
        pltpu.VMEM_SHARED((_NP, _D), jnp.float32),
        pltpu.SemaphoreType.DMA,
    ]
    if with_ea:
        out_type.append(jax.ShapeDtypeStruct((_NC, _NP, _DEP), jnp.float32))
        scratch += [
            pltpu.VMEM((_CHUNK, _DEP), jnp.float32),
            pltpu.VMEM_SHARED((_NP, _DEP), jnp.float32),
        ]

    mesh = plsc.VectorSubcoreMesh(core_axis_name="c", subcore_axis_name="s")

    def body(*refs):
        if with_ea:
            (x_hbm, src_hbm, dst_hbm, ea_hbm, zx_hbm, zea_hbm,
             acc_out, ea_out,
             sidx_v, didx_v, rows_v, acc_sp, sem, ea_v, ea_sp) = refs
        else:
            (x_hbm, src_hbm, dst_hbm, zx_hbm,
             acc_out,
             sidx_v, didx_v, rows_v, acc_sp, sem) = refs

        cid = lax.axis_index("c")
        sid = lax.axis_index("s")
        wid = sid * _NC + cid
        base = wid * _EPW

        pltpu.sync_copy(zx_hbm, acc_sp.at[pl.ds(sid * _RPT, _RPT)])
        if with_ea:
            pltpu.sync_copy(zea_hbm, ea_sp.at[pl.ds(sid * _RPT, _RPT)])
        plsc.subcore_barrier()

        def step(j, carry):
            off = pl.multiple_of(base + j * _CHUNK, 8)
            pltpu.sync_copy(src_hbm.at[pl.ds(off, _CHUNK)], sidx_v)
            pltpu.sync_copy(dst_hbm.at[pl.ds(off, _CHUNK)], didx_v)
            pltpu.async_copy(x_hbm.at[sidx_v], rows_v, sem).wait()
            pltpu.sync_copy(rows_v, acc_sp.at[didx_v], add=True)
            if with_ea:
                pltpu.sync_copy(ea_hbm.at[pl.ds(off, _CHUNK)], ea_v)
                pltpu.sync_copy(ea_v, ea_sp.at[didx_v], add=True)
            return carry

        lax.fori_loop(0, _NCHUNK, step, 0)
        plsc.subcore_barrier()

        pltpu.sync_copy(acc_sp.at[pl.ds(sid * _RPT, _RPT)],
                        acc_out.at[cid, pl.ds(sid * _RPT, _RPT)])
        if with_ea:
            pltpu.sync_copy(ea_sp.at[pl.ds(sid * _RPT, _RPT)],
                            ea_out.at[cid, pl.ds(sid * _RPT, _RPT)])

    return pl.kernel(
        body, mesh=mesh, out_type=out_type, scratch_types=scratch,
        compiler_params=pltpu.CompilerParams(use_tc_tiling_on_sc=False))


@functools.cache
def _sc_pass(with_ea):
    return _sc_edge_pass(with_ea)


def _onehot_and_invcnt(batch2d):
    gids = lax.broadcasted_iota(jnp.int32, (1, _G), 1)
    onehot = (batch2d == gids).astype(jnp.float32)
    ones = jnp.ones((_N, 1), jnp.float32)
    cnt = lax.dot_general(onehot, ones, (((0,), (0,)), ((), ())))
    return onehot, 1.0 / jnp.maximum(cnt, 1.0)


def _tc_conv1_body(x_ref, s1_ref, batch_ref, w1_ref, out_ref):
    x = x_ref[...]
    out = x + s1_ref[0, :_N] + s1_ref[1, :_N]
    onehot, invcnt = _onehot_and_invcnt(batch_ref[...])
    pooled = lax.dot_general(onehot, out, (((0,), (0,)), ((), ())))
    vmsg = pooled * invcnt
    out = out + lax.dot_general(onehot, vmsg, (((1,), (0,)), ((), ())))
    out = jnp.maximum(lax.dot_general(out, w1_ref[...],
                                      (((1,), (0,)), ((), ()))), 0.0)
    out_ref[...] = out + x


_tc_conv1 = pl.pallas_call(
    _tc_conv1_body,
    out_shape=jax.ShapeDtypeStruct((_N, _D), jnp.float32),
)


def _tc_conv2_body(x1_ref, s2_ref, ea_ref, batch_ref, wea_ref,
                   wn1_ref, bn1_ref, wn2_ref, bn2_ref, wfc_ref, bfc_ref,
                   out_ref):
    x1 = x1_ref[...]
    ea = ea_ref[0, :_N] + ea_ref[1, :_N]
    agg = s2_ref[0, :_N] + s2_ref[1, :_N] + lax.dot_general(
        ea, wea_ref[...], (((1,), (0,)), ((), ())))
    h = jnp.maximum(lax.dot_general(agg, wn1_ref[...],
                                    (((1,), (0,)), ((), ()))) + bn1_ref[...],
                    0.0)
    out2 = lax.dot_general(h, wn2_ref[...],
                           (((1,), (0,)), ((), ()))) + bn2_ref[...]
    x2 = out2 + x1
    onehot, invcnt = _onehot_and_invcnt(batch_ref[...])
    pooled = lax.dot_general(onehot, x2, (((0,), (0,)), ((), ()))) * invcnt
    out_ref[...] = lax.dot_general(pooled, wfc_ref[...],
                                   (((1,), (0,)), ((), ()))) + bfc_ref[...]


_tc_conv2 = pl.pallas_call(
    _tc_conv2_body,
    out_shape=jax.ShapeDtypeStruct((_G, _D), jnp.float32),
)


def kernel(x, edge_index, edge_attr, batch, W1, We, be, Wn1, bn1, Wn2, bn2,
           Wfc, bfc):
    src = edge_index[0]
    dst = edge_index[1]
    ea_aug = jnp.concatenate(
        [edge_attr,
         jnp.ones((_E, 1), jnp.float32),
         jnp.zeros((_E, _DEP - 17), jnp.float32)], axis=1)
    wea = jnp.concatenate(
        [We, be[None, :], jnp.zeros((_DEP - 17, _D), jnp.float32)], axis=0)
    zx = jnp.zeros((_RPT, _D), jnp.float32)
    zea = jnp.zeros((_RPT, _DEP), jnp.float32)
    batch2d = batch[:, None]

    s1p, eap = _sc_pass(True)(x, src, dst, ea_aug, zx, zea)
    x1 = _tc_conv1(x, s1p, batch2d, W1)
    s2p = _sc_pass(False)(x1, src, dst, zx)
    if isinstance(s2p, (list, tuple)):
        (s2p,) = s2p
    return _tc_conv2(x1, s2p, eap, batch2d, wea,
                     Wn1, bn1[None, :], Wn2, bn2[None, :], Wfc, bfc[None, :])


# speedup vs baseline: 7.4020x; 2.0056x over previous
import functools

import jax
import jax.numpy as jnp
from jax import lax
from jax.experimental import pallas as pl
from jax.experimental.pallas import tpu as pltpu
from jax.experimental.pallas import tpu_sc as plsc

_N = 10000
_E = 320000
_D = 128
_DE = 16
_G = 64

_NC = 2
_NS = 16
_NW = _NC * _NS
_EPW = _E // _NW
_CHUNK = 100
_NCHUNK = _EPW // _CHUNK
_NB = 2
_BLK = 50
_NBLK = _NCHUNK // _BLK
_NP = 10112
_RPT = _NP // _NS


def _sc_edge_pass(with_ea):
    out_type = [jax.ShapeDtypeStruct((_NC, _NP, _D), jnp.float32)]
    scratch = [
        pltpu.VMEM((_BLK, _CHUNK), jnp.int32),
        pltpu.VMEM((_BLK, _CHUNK), jnp.int32),
        [pltpu.VMEM((_CHUNK, _D), jnp.float32) for _ in range(_NB)],
        pltpu.VMEM_SHARED((_NP, _D), jnp.float32),
        [pltpu.SemaphoreType.DMA for _ in range(_NB)],
    ]
    if with_ea:
        out_type.append(jax.ShapeDtypeStruct((_NC, _NP, _DE), jnp.float32))
        scratch += [
            [pltpu.VMEM((_CHUNK, _DE), jnp.float32) for _ in range(_NB)],
            pltpu.VMEM_SHARED((_NP, _DE), jnp.float32),
            [pltpu.SemaphoreType.DMA for _ in range(_NB)],
        ]

    mesh = plsc.VectorSubcoreMesh(core_axis_name="c", subcore_axis_name="s")

    def body(*refs):
        if with_ea:
            (x_hbm, src_hbm, dst_hbm, ea_hbm, zx_hbm, zea_hbm,
             acc_out, ea_out,
             sidx_v, didx_v, rows_v, acc_sp, gsem, ea_v, ea_sp, esem) = refs
        else:
            (x_hbm, src_hbm, dst_hbm, zx_hbm,
             acc_out,
             sidx_v, didx_v, rows_v, acc_sp, gsem) = refs

        cid = lax.axis_index("c")
        sid = lax.axis_index("s")
        wid = sid * _NC + cid

        pltpu.sync_copy(zx_hbm, acc_sp.at[pl.ds(sid * _RPT, _RPT)])
        if with_ea:
            pltpu.sync_copy(zea_hbm, ea_sp.at[pl.ds(sid * _RPT, _RPT)])
        plsc.subcore_barrier()

        def run_block(blk):
            pltpu.sync_copy(src_hbm.at[wid, pl.ds(blk * _BLK, _BLK)], sidx_v)
            pltpu.sync_copy(dst_hbm.at[wid, pl.ds(blk * _BLK, _BLK)], didx_v)

            def issue(r, b):
                pltpu.async_copy(x_hbm.at[sidx_v.at[r]], rows_v[b], gsem[b])
                if with_ea:
                    pltpu.async_copy(ea_hbm.at[wid, blk * _BLK + r],
                                     ea_v[b], esem[b])

            for b in range(_NB):
                issue(b, b)

            def outer(rr, carry):
                for b in range(_NB):
                    r = rr * _NB + b
                    pltpu.make_async_copy(
                        x_hbm.at[sidx_v.at[r]], rows_v[b], gsem[b]).wait()
                    pltpu.sync_copy(rows_v[b], acc_sp.at[didx_v.at[r]],
                                    add=True)
                    if with_ea:
                        pltpu.make_async_copy(
                            ea_hbm.at[wid, blk * _BLK + r],
                            ea_v[b], esem[b]).wait()
                        pltpu.sync_copy(ea_v[b], ea_sp.at[didx_v.at[r]],
                                        add=True)

                    @pl.when(r + _NB < _BLK)
                    def _():
                        issue(r + _NB, b)
                return carry

            lax.fori_loop(0, _BLK // _NB, outer, 0)

        for blk in range(_NBLK):
            run_block(blk)
        plsc.subcore_barrier()

        pltpu.sync_copy(acc_sp.at[pl.ds(sid * _RPT, _RPT)],
                        acc_out.at[cid, pl.ds(sid * _RPT, _RPT)])
        if with_ea:
            pltpu.sync_copy(ea_sp.at[pl.ds(sid * _RPT, _RPT)],
                            ea_out.at[cid, pl.ds(sid * _RPT, _RPT)])

    return pl.kernel(
        body, mesh=mesh, out_type=out_type, scratch_types=scratch,
        compiler_params=pltpu.CompilerParams(use_tc_tiling_on_sc=False))


@functools.cache
def _sc_pass(with_ea):
    return _sc_edge_pass(with_ea)


def _onehot_and_invcnt(batch2d):
    gids = lax.broadcasted_iota(jnp.int32, (1, _G), 1)
    onehot = (batch2d == gids).astype(jnp.float32)
    ones = jnp.ones((_N, 1), jnp.float32)
    cnt = lax.dot_general(onehot, ones, (((0,), (0,)), ((), ())))
    return onehot, 1.0 / jnp.maximum(cnt, 1.0)


def _tc_conv1_body(x_ref, s1_ref, batch_ref, w1_ref, be_ref,
                   out_ref, outb_ref):
    x = x_ref[...]
    out = x + s1_ref[0, :_N] + s1_ref[1, :_N]
    onehot, invcnt = _onehot_and_invcnt(batch_ref[...])
    pooled = lax.dot_general(onehot, out, (((0,), (0,)), ((), ())))
    vmsg = pooled * invcnt
    out = out + lax.dot_general(onehot, vmsg, (((1,), (0,)), ((), ())))
    out = jnp.maximum(lax.dot_general(out, w1_ref[...],
                                      (((1,), (0,)), ((), ()))), 0.0)
    x1 = out + x
    out_ref[...] = x1
    outb_ref[...] = x1 + be_ref[...]


_tc_conv1 = pl.pallas_call(
    _tc_conv1_body,
    out_shape=[jax.ShapeDtypeStruct((_N, _D), jnp.float32),
               jax.ShapeDtypeStruct((_N, _D), jnp.float32)],
)


def _tc_conv2_body(x1_ref, s2_ref, ea_ref, batch_ref, we_ref,
                   wn1_ref, bn1_ref, wn2_ref, bn2_ref, wfc_ref, bfc_ref,
                   out_ref):
    x1 = x1_ref[...]
    ea = ea_ref[0, :_N] + ea_ref[1, :_N]
    agg = s2_ref[0, :_N] + s2_ref[1, :_N] + lax.dot_general(
        ea, we_ref[...], (((1,), (0,)), ((), ())))
    h = jnp.maximum(lax.dot_general(agg, wn1_ref[...],
                                    (((1,), (0,)), ((), ()))) + bn1_ref[...],
                    0.0)
    out2 = lax.dot_general(h, wn2_ref[...],
                           (((1,), (0,)), ((), ()))) + bn2_ref[...]
    x2 = out2 + x1
    onehot, invcnt = _onehot_and_invcnt(batch_ref[...])
    pooled = lax.dot_general(onehot, x2, (((0,), (0,)), ((), ()))) * invcnt
    out_ref[...] = lax.dot_general(pooled, wfc_ref[...],
                                   (((1,), (0,)), ((), ()))) + bfc_ref[...]


_tc_conv2 = pl.pallas_call(
    _tc_conv2_body,
    out_shape=jax.ShapeDtypeStruct((_G, _D), jnp.float32),
)


def kernel(x, edge_index, edge_attr, batch, W1, We, be, Wn1, bn1, Wn2, bn2,
           Wfc, bfc):
    src = edge_index[0].reshape(_NW, _NCHUNK, _CHUNK)
    dst = edge_index[1].reshape(_NW, _NCHUNK, _CHUNK)
    ea = edge_attr.reshape(_NW, _NCHUNK, _CHUNK, _DE)
    zx = jnp.zeros((_RPT, _D), jnp.float32)
    zea = jnp.zeros((_RPT, _DE), jnp.float32)
    batch2d = batch[:, None]

    s1p, eap = _sc_pass(True)(x, src, dst, ea, zx, zea)
    x1, x1b = _tc_conv1(x, s1p, batch2d, W1, be[None, :])
    s2p = _sc_pass(False)(x1b, src, dst, zx)
    if isinstance(s2p, (list, tuple)):
        (s2p,) = s2p
    return _tc_conv2(x1, s2p, eap, batch2d, We,
                     Wn1, bn1[None, :], Wn2, bn2[None, :], Wfc, bfc[None, :])
